# Initial kernel scaffold; baseline (speedup 1.0000x reference)
import jax
import jax.numpy as jnp
from jax.experimental import pallas as pl


def kernel(gram2, gram3, word, uniblock, E2, E3, Ew, Eu, W1, b1, W2, b2):
    raise NotImplementedError("write your pallas kernel here")


# speedup vs baseline: 2.6719x; 2.6719x over previous
import functools

import jax
import jax.numpy as jnp
from jax import lax
from jax.experimental import pallas as pl
from jax.experimental.pallas import tpu as pltpu
from jax.experimental.pallas import tpu_sc as plsc

B = 4096
L = 50
D2 = 32
DW = 64
DU = 16
HDIM = 512
N_LANG = 235

NC = 2
NS = 16
NW = NC * NS
EXW = B // NW
CE = 2
R = CE * L
NCHUNK = EXW // CE


def _sc_pool_body(g2_hbm, g3_hbm, w_hbm, uni_hbm, e2_hbm, e3_hbm, ew_hbm,
                  eu_hbm, s2_hbm, s3_hbm, sw_hbm, pu_hbm,
                  idx_v, bufg0, bufg1, bufw0, bufw1,
                  acc2, acc3, accw, uidx_v, ubuf, sem0, sem1):
    wid = lax.axis_index("s") * NC + lax.axis_index("c")
    base = wid * EXW

    def run_table(idx_src_hbm, table_hbm, out_hbm, acc, bufs, d):
        nvec = d // 16
        sems = (sem0, sem1)
        pltpu.sync_copy(idx_src_hbm.at[wid], idx_v)

        def start(c, b):
            pltpu.async_copy(table_hbm.at[idx_v.at[c]], bufs[b], sems[b])

        def wait(b):
            pltpu.make_async_copy(
                table_hbm.at[idx_v.at[0]], bufs[b], sems[b]).wait()

        def accum(c, b):
            buf = bufs[b]
            for e in range(CE):
                def lbody(l, carry):
                    return tuple(cv + buf[e * L + l, pl.ds(16 * dd, 16)]
                                 for dd, cv in enumerate(carry))
                r = lax.fori_loop(
                    0, L, lbody,
                    tuple(jnp.zeros((16,), jnp.float32) for _ in range(nvec)),
                    unroll=5)
                for dd in range(nvec):
                    acc[c * CE + e, pl.ds(16 * dd, 16)] = r[dd]

        start(0, 0)

        def outer(c2, carry):
            for bb in range(2):
                c = 2 * c2 + bb

                @pl.when(c < NCHUNK - 1)
                def _():
                    start(c + 1, 1 - bb)

                wait(bb)
                accum(c, bb)
            return carry

        lax.fori_loop(0, NCHUNK // 2, outer, 0)
        pltpu.sync_copy(acc, out_hbm.at[pl.ds(base, EXW)])

    run_table(g2_hbm, e2_hbm, s2_hbm, acc2, (bufg0, bufg1), D2)
    run_table(g3_hbm, e3_hbm, s3_hbm, acc3, (bufg0, bufg1), D2)
    run_table(w_hbm, ew_hbm, sw_hbm, accw, (bufw0, bufw1), DW)

    pltpu.sync_copy(uni_hbm.at[pl.ds(base, EXW)], uidx_v)
    pltpu.async_copy(eu_hbm.at[uidx_v], ubuf, sem0).wait()
    pltpu.sync_copy(ubuf, pu_hbm.at[pl.ds(base, EXW)])


@functools.cache
def _build_sc_pool():
    return functools.partial(
        pl.kernel,
        out_type=[
            jax.ShapeDtypeStruct((B, D2), jnp.float32),
            jax.ShapeDtypeStruct((B, D2), jnp.float32),
            jax.ShapeDtypeStruct((B, DW), jnp.float32),
            jax.ShapeDtypeStruct((B, DU), jnp.float32),
        ],
        mesh=plsc.VectorSubcoreMesh(core_axis_name="c", subcore_axis_name="s",
                                    num_cores=NC, num_subcores=NS),
        compiler_params=pltpu.CompilerParams(use_tc_tiling_on_sc=False),
        scratch_types=[
            pltpu.VMEM((NCHUNK, R), jnp.int32),
            pltpu.VMEM((R, D2), jnp.float32),
            pltpu.VMEM((R, D2), jnp.float32),
            pltpu.VMEM((R, DW), jnp.float32),
            pltpu.VMEM((R, DW), jnp.float32),
            pltpu.VMEM((EXW, D2), jnp.float32),
            pltpu.VMEM((EXW, D2), jnp.float32),
            pltpu.VMEM((EXW, DW), jnp.float32),
            pltpu.VMEM((EXW,), jnp.int32),
            pltpu.VMEM((EXW, DU), jnp.float32),
            pltpu.SemaphoreType.DMA,
            pltpu.SemaphoreType.DMA,
        ],
    )(_sc_pool_body)


BM = 512
GB = B // BM


def _mlp_body(g2_ref, g3_ref, w_ref, s2_ref, s3_ref, pu_ref, sw_ref,
              e20_ref, e30_ref, ew0_ref, w1a_ref, w1b_ref, w1c_ref, w1d_ref,
              b1_ref, w2_ref, b2_ref, out_ref):
    f32 = jnp.float32

    def pool(s_ref, g_ref, e0_ref):
        cnt = jnp.sum((g_ref[...] != 0).astype(f32), axis=1, keepdims=True)
        return (s_ref[...] - (float(L) - cnt) * e0_ref[...]) / jnp.maximum(
            cnt, 1.0)

    p2 = pool(s2_ref, g2_ref, e20_ref)
    p3 = pool(s3_ref, g3_ref, e30_ref)
    pw = pool(sw_ref, w_ref, ew0_ref)
    h = (jnp.dot(p2, w1a_ref[...], preferred_element_type=f32)
         + jnp.dot(p3, w1b_ref[...], preferred_element_type=f32)
         + jnp.dot(pu_ref[...], w1c_ref[...], preferred_element_type=f32)
         + jnp.dot(pw, w1d_ref[...], preferred_element_type=f32)
         + b1_ref[...])
    h = jnp.maximum(h, 0.0)
    out_ref[...] = jnp.dot(h, w2_ref[...], preferred_element_type=f32) \
        + b2_ref[...]


def _mlp(g2i, g3i, wdi, s2, s3, pu, sw, e20, e30, ew0,
         w1a, w1b, w1c, w1d, b1r, w2, b2r):
    def bspec(c):
        return pl.BlockSpec((BM, c), lambda i: (i, 0))

    def fspec(r, c):
        return pl.BlockSpec((r, c), lambda i: (0, 0))

    return pl.pallas_call(
        _mlp_body,
        grid=(GB,),
        in_specs=[
            bspec(L), bspec(L), bspec(L),
            bspec(D2), bspec(D2), bspec(DU), bspec(DW),
            fspec(1, D2), fspec(1, D2), fspec(1, DW),
            fspec(D2, HDIM), fspec(D2, HDIM), fspec(DU, HDIM),
            fspec(DW, HDIM), fspec(1, HDIM),
            fspec(HDIM, N_LANG), fspec(1, N_LANG),
        ],
        out_specs=pl.BlockSpec((BM, N_LANG), lambda i: (i, 0)),
        out_shape=jax.ShapeDtypeStruct((B, N_LANG), jnp.float32),
    )(g2i, g3i, wdi, s2, s3, pu, sw, e20, e30, ew0,
      w1a, w1b, w1c, w1d, b1r, w2, b2r)


def kernel(gram2, gram3, word, uniblock, E2, E3, Ew, Eu, W1, b1, W2, b2):
    g2i = gram2.astype(jnp.int32)
    g3i = gram3.astype(jnp.int32)
    wdi = word.astype(jnp.int32)
    ubi = uniblock.astype(jnp.int32)

    s2, s3, sw, pu = _build_sc_pool()(
        g2i.reshape(NW, NCHUNK, R),
        g3i.reshape(NW, NCHUNK, R),
        wdi.reshape(NW, NCHUNK, R),
        ubi, E2, E3, Ew, Eu)

    return _mlp(
        g2i, g3i, wdi, s2, s3, pu, sw,
        E2[0:1], E3[0:1], Ew[0:1],
        W1[0:D2], W1[D2:2 * D2], W1[2 * D2:2 * D2 + DU], W1[2 * D2 + DU:],
        b1.reshape(1, HDIM), W2, b2.reshape(1, N_LANG))
